# baseline (speedup 1.0000x reference)
import functools

import jax
import jax.numpy as jnp
from jax import lax
from jax.experimental import pallas as pl
from jax.experimental.pallas import tpu as pltpu
from jax.experimental.pallas import tpu_sc as plsc

_NUM_EMB = 1000000
_D = 32
_B = 16384
_S = 50
_TOTAL = _B * _S
_NW = 32
_CH = 512
_PER_W = _TOTAL // _NW
_CHUNKS = _PER_W // _CH
_CPS = _B // _CH

_mesh = plsc.VectorSubcoreMesh(core_axis_name="c", subcore_axis_name="s")


@functools.partial(
    pl.kernel,
    mesh=_mesh,
    compiler_params=pltpu.CompilerParams(
        use_tc_tiling_on_sc=False, needs_layout_passes=False
    ),
    out_type=jax.ShapeDtypeStruct((_S, _D, _B), jnp.float32),
    scratch_types=[
        pltpu.VMEM((_PER_W,), jnp.int32),
        pltpu.VMEM((2, _CH, _D), jnp.float32),
        pltpu.VMEM((2, _D, _CH + 1), jnp.float32),
        pltpu.SemaphoreType.DMA,
        pltpu.SemaphoreType.DMA,
        pltpu.SemaphoreType.DMA,
    ],
)
def _emb_lookup(table_hbm, idx_hbm, out_hbm, idx_v, rows_v, t_v,
                sem_g, sem_s0, sem_s1):
    wid = lax.axis_index("s") * 2 + lax.axis_index("c")
    base = wid * _PER_W
    pltpu.sync_copy(idx_hbm.at[pl.ds(base, _PER_W)], idx_v)
    c0 = wid * _CHUNKS
    sems = (sem_s0, sem_s1)
    lane = lax.iota(jnp.int32, 16)
    d_lo = lane
    d_hi = lane + 16

    def drain_out(b):
        pltpu.make_async_copy(
            t_v.at[b, :, pl.ds(0, _CH)], out_hbm.at[0, :, pl.ds(0, _CH)], sems[b]
        ).wait()

    def fire_gather(i, b):
        return pltpu.async_copy(
            table_hbm.at[idx_v.at[pl.ds(i * _CH, _CH)]], rows_v.at[b], sem_g
        )

    def transpose_and_store(i, b):
        def tr(k, carry):
            for u in range(16):
                tok = k * 16 + u
                lo = rows_v[b, tok, pl.ds(0, 16)]
                hi = rows_v[b, tok, pl.ds(16, 16)]
                ts = jnp.full((16,), tok, jnp.int32)
                plsc.store_scatter(t_v.at[b], [d_lo, ts], lo)
                plsc.store_scatter(t_v.at[b], [d_hi, ts], hi)
            return carry

        lax.fori_loop(0, _CH // 16, tr, 0)
        c = c0 + i
        s = c // _CPS
        t0 = (c % _CPS) * _CH
        pltpu.async_copy(
            t_v.at[b, :, pl.ds(0, _CH)], out_hbm.at[s, :, pl.ds(t0, _CH)], sems[b]
        )

    def do_pair(t, drain):
        if drain:
            drain_out(0)
        g0 = fire_gather(2 * t, 0)
        if drain:
            drain_out(1)
        g1 = fire_gather(2 * t + 1, 1)
        g0.wait()
        transpose_and_store(2 * t, 0)
        g1.wait()
        transpose_and_store(2 * t + 1, 1)

    do_pair(0, False)

    def pair(t, carry):
        do_pair(t, True)
        return carry

    lax.fori_loop(1, _CHUNKS // 2, pair, 0)
    drain_out(0)
    drain_out(1)


def kernel(token_ids, weight):
    idx = jnp.transpose(token_ids).reshape(_TOTAL).astype(jnp.int32)
    wt = weight * jnp.float32(1.0000001)
    out = _emb_lookup(wt, idx)
    return jnp.transpose(out, (2, 0, 1))


# speedup vs baseline: 1.4139x; 1.4139x over previous
import functools

import jax
import jax.numpy as jnp
from jax import lax
from jax.experimental import pallas as pl
from jax.experimental.pallas import tpu as pltpu
from jax.experimental.pallas import tpu_sc as plsc

_NUM_EMB = 1000000
_D = 32
_B = 16384
_S = 50
_TOTAL = _B * _S
_NW = 32
_CH = 512
_PER_W = _TOTAL // _NW
_CHUNKS = _PER_W // _CH
_CPS = _B // _CH

_mesh = plsc.VectorSubcoreMesh(core_axis_name="c", subcore_axis_name="s")


@functools.partial(
    pl.kernel,
    mesh=_mesh,
    compiler_params=pltpu.CompilerParams(
        use_tc_tiling_on_sc=False, needs_layout_passes=False
    ),
    out_type=jax.ShapeDtypeStruct((_S, _D, _B), jnp.float32),
    scratch_types=[
        pltpu.VMEM((_PER_W,), jnp.int32),
        pltpu.VMEM((2, _CH, _D), jnp.float32),
        pltpu.VMEM((2, _D, _CH + 1), jnp.float32),
        pltpu.SemaphoreType.DMA,
        pltpu.SemaphoreType.DMA,
        pltpu.SemaphoreType.DMA,
    ],
)
def _emb_lookup(table_hbm, idx_hbm, out_hbm, idx_v, rows_v, t_v,
                sem_g, sem_s0, sem_s1):
    wid = lax.axis_index("s") * 2 + lax.axis_index("c")
    base = wid * _PER_W
    pltpu.sync_copy(idx_hbm.at[pl.ds(base, _PER_W)], idx_v)
    c0 = wid * _CHUNKS
    sems = (sem_s0, sem_s1)
    lane = lax.iota(jnp.int32, 16)
    d_lo = lane
    d_hi = lane + 16

    def drain_out(b):
        pltpu.make_async_copy(
            t_v.at[b, :, pl.ds(0, _CH)], out_hbm.at[0, :, pl.ds(0, _CH)], sems[b]
        ).wait()

    def fire_gather(i, b):
        return pltpu.async_copy(
            table_hbm.at[idx_v.at[pl.ds(i * _CH, _CH)]], rows_v.at[b], sem_g
        )

    def transpose_and_store(i, b):
        def tr(k, carry):
            for u in range(16):
                tok = k * 16 + u
                lo = rows_v[b, tok, pl.ds(0, 16)]
                hi = rows_v[b, tok, pl.ds(16, 16)]
                ts = jnp.full((16,), tok, jnp.int32)
                plsc.store_scatter(t_v.at[b], [d_lo, ts], lo)
                plsc.store_scatter(t_v.at[b], [d_hi, ts], hi)
            return carry

        lax.fori_loop(0, _CH // 16, tr, 0)
        c = c0 + i
        s = c // _CPS
        t0 = (c % _CPS) * _CH
        pltpu.async_copy(
            t_v.at[b, :, pl.ds(0, _CH)], out_hbm.at[s, :, pl.ds(t0, _CH)], sems[b]
        )

    def do_pair(t, drain):
        if drain:
            drain_out(0)
        g0 = fire_gather(2 * t, 0)
        if drain:
            drain_out(1)
        g1 = fire_gather(2 * t + 1, 1)
        g0.wait()
        transpose_and_store(2 * t, 0)
        g1.wait()
        transpose_and_store(2 * t + 1, 1)

    do_pair(0, False)

    def pair(t, carry):
        do_pair(t, True)
        return carry

    lax.fori_loop(1, _CHUNKS // 2, pair, 0)
    drain_out(0)
    drain_out(1)


def kernel(token_ids, weight):
    idx = jnp.transpose(token_ids).reshape(_TOTAL).astype(jnp.int32)
    out = _emb_lookup(weight, idx)
    return jnp.transpose(out, (2, 0, 1))
